# Initial kernel scaffold; baseline (speedup 1.0000x reference)
import jax
import jax.numpy as jnp
from jax.experimental import pallas as pl


def kernel(x, weight):
    raise NotImplementedError("write your pallas kernel here")


# speedup vs baseline: 1.4376x; 1.4376x over previous
import functools

import jax
import jax.numpy as jnp
from jax import lax
from jax.experimental import pallas as pl
from jax.experimental.pallas import tpu as pltpu
from jax.experimental.pallas import tpu_sc as plsc

EMB_DIM = 32

NUM_WORKERS = 32
CHUNK = 128


def _build(batch, fields):
    b_total = batch * fields
    assert b_total % (NUM_WORKERS * CHUNK) == 0
    b_per_w = b_total // NUM_WORKERS
    n_chunk = b_per_w // CHUNK

    mesh = plsc.VectorSubcoreMesh(core_axis_name="c", subcore_axis_name="s")

    @functools.partial(
        pl.kernel,
        mesh=mesh,
        out_type=jax.ShapeDtypeStruct((b_total, EMB_DIM), jnp.float32),
        scratch_types=[
            pltpu.VMEM((n_chunk, CHUNK), jnp.int32),
            pltpu.VMEM((CHUNK, EMB_DIM), jnp.float32),
            pltpu.SemaphoreType.DMA,
        ],
        compiler_params=pltpu.CompilerParams(use_tc_tiling_on_sc=False),
    )
    def emb(table_hbm, idx_hbm, out_hbm, idx_v, rows_v, gsem):
        wid = lax.axis_index("s") * 2 + lax.axis_index("c")
        base = wid * b_per_w
        pltpu.sync_copy(idx_hbm.at[wid], idx_v)

        def body(j, carry):
            pltpu.async_copy(table_hbm.at[idx_v.at[j]], rows_v, gsem).wait()
            pltpu.sync_copy(rows_v, out_hbm.at[pl.ds(base + j * CHUNK, CHUNK)])
            return carry

        lax.fori_loop(0, n_chunk, body, 0)

    return emb


def kernel(x, weight):
    batch, fields = x.shape
    b_per_w = batch * fields // NUM_WORKERS
    idx = x.reshape(NUM_WORKERS, b_per_w // CHUNK, CHUNK)
    out = _build(batch, fields)(weight, idx)
    return out.reshape(batch, fields, EMB_DIM)


# speedup vs baseline: 1.5757x; 1.0961x over previous
import functools

import jax
import jax.numpy as jnp
from jax import lax
from jax.experimental import pallas as pl
from jax.experimental.pallas import tpu as pltpu
from jax.experimental.pallas import tpu_sc as plsc

EMB_DIM = 32

NUM_WORKERS = 32
CHUNK = 128
NSLOT = 8
LA = 4


def _build(batch, fields):
    b_total = batch * fields
    assert b_total % (NUM_WORKERS * CHUNK) == 0
    b_per_w = b_total // NUM_WORKERS
    n_chunk = b_per_w // CHUNK
    assert n_chunk % NSLOT == 0 and n_chunk >= 2 * NSLOT

    mesh = plsc.VectorSubcoreMesh(core_axis_name="c", subcore_axis_name="s")

    @functools.partial(
        pl.kernel,
        mesh=mesh,
        out_type=jax.ShapeDtypeStruct((b_total, EMB_DIM), jnp.float32),
        scratch_types=[
            pltpu.VMEM((n_chunk, CHUNK), jnp.int32),
            pltpu.VMEM((NSLOT, CHUNK, EMB_DIM), jnp.float32),
            pltpu.SemaphoreType.DMA((NSLOT,)),
            pltpu.SemaphoreType.DMA((NSLOT,)),
        ],
        compiler_params=pltpu.CompilerParams(use_tc_tiling_on_sc=False),
    )
    def emb(table_hbm, idx_hbm, out_hbm, idx_v, rows_v, gsem, osem):
        wid = lax.axis_index("s") * 2 + lax.axis_index("c")
        base = wid * b_per_w
        pltpu.sync_copy(idx_hbm.at[wid], idx_v)

        def gstart(c, slot):
            pltpu.make_async_copy(
                table_hbm.at[idx_v.at[c]], rows_v.at[slot], gsem.at[slot]
            ).start()

        def gwait(slot):
            pltpu.make_async_copy(
                table_hbm.at[idx_v.at[0]], rows_v.at[slot], gsem.at[slot]
            ).wait()

        def ostart(c, slot):
            pltpu.make_async_copy(
                rows_v.at[slot], out_hbm.at[pl.ds(base + c * CHUNK, CHUNK)],
                osem.at[slot],
            ).start()

        def owait(slot):
            pltpu.make_async_copy(
                rows_v.at[slot], out_hbm.at[pl.ds(base, CHUNK)], osem.at[slot]
            ).wait()

        for c in range(LA):
            gstart(c, c)
        for j in range(NSLOT):
            a = j + LA
            if a >= NSLOT:
                owait(a % NSLOT)
            gstart(a, a % NSLOT)
            gwait(j % NSLOT)
            ostart(j, j % NSLOT)

        def group(gi, carry):
            j0 = gi * NSLOT
            for b in range(NSLOT):
                j = j0 + b
                a = j + LA
                owait((b + LA) % NSLOT)
                gstart(a, (b + LA) % NSLOT)
                gwait(b)
                ostart(j, b)
            return carry

        lax.fori_loop(1, n_chunk // NSLOT - 1, group, 0)

        for j in range(n_chunk - NSLOT, n_chunk):
            a = j + LA
            if a < n_chunk:
                owait(a % NSLOT)
                gstart(a, a % NSLOT)
            gwait(j % NSLOT)
            ostart(j, j % NSLOT)
        for s in range(NSLOT):
            owait(s)

    return emb


def kernel(x, weight):
    batch, fields = x.shape
    b_per_w = batch * fields // NUM_WORKERS
    idx = x.reshape(NUM_WORKERS, b_per_w // CHUNK, CHUNK)
    out = _build(batch, fields)(weight, idx)
    return out.reshape(batch, fields, EMB_DIM)
